# Initial kernel scaffold; baseline (speedup 1.0000x reference)
import jax
import jax.numpy as jnp
from jax.experimental import pallas as pl


def kernel(x, shifts):
    raise NotImplementedError("write your pallas kernel here")


# speedup vs baseline: 22.1080x; 22.1080x over previous
import functools

import jax
import jax.numpy as jnp
from jax import lax
from jax.experimental import pallas as pl
from jax.experimental.pallas import tpu as pltpu
from jax.experimental.pallas import tpu_sc as plsc

_NSHIFT = 2
_L = 16
_NC = 2
_NS = 16
_NW = _NC * _NS
_BLK = 8


def _phase_shuffle_sc(x, shifts_i32, interpret=False):
    B, C, T = x.shape
    rows_per_w = (B * C) // _NW
    nvec = T // _L
    nblk = (nvec - 8) // _BLK

    mesh = plsc.VectorSubcoreMesh(core_axis_name="c", subcore_axis_name="s")

    @functools.partial(
        pl.kernel,
        out_type=jax.ShapeDtypeStruct((B, C, T), jnp.float32),
        mesh=mesh,
        interpret=interpret,
        compiler_params=pltpu.CompilerParams(needs_layout_passes=False),
        scratch_types=[pltpu.VMEM((T,), jnp.float32)] * 4
        + [pltpu.VMEM((B,), jnp.int32)]
        + [pltpu.SemaphoreType.DMA] * 4,
    )
    def body(x_hbm, sh_hbm, out_hbm, *scratch):
        in_b = scratch[0:2]
        out_b = scratch[2:4]
        sh_v = scratch[4]
        in_sems = scratch[5:7]
        out_sems = scratch[7:9]
        wid = lax.axis_index("s") * _NC + lax.axis_index("c")
        pltpu.sync_copy(sh_hbm, sh_v)
        row0 = wid * rows_per_w
        iota = lax.iota(jnp.int32, _L)

        def in_copy(r, j):
            return pltpu.make_async_copy(
                x_hbm.at[r // C, r % C], in_b[j], in_sems[j])

        def out_copy(r, j):
            return pltpu.make_async_copy(
                out_b[j], out_hbm.at[r // C, r % C], out_sems[j])

        in_copy(row0, 0).start()
        in_copy(row0 + 1, 1).start()

        def do_row(r, loc, j):
            src = in_b[j]
            dst = out_b[j]
            in_copy(r, j).wait()

            @pl.when(loc >= 2)
            def _():
                out_copy(r - 2, j).wait()

            kv = plsc.load_gather(sh_v, [jnp.full((_L,), r // C, jnp.int32)])
            base = iota - (kv - _NSHIFT)
            idx0 = jnp.where(base < 0, -base, base)
            hvals = [plsc.load_gather(src, [idx0])] + [
                plsc.load_gather(src, [base + v * _L]) for v in (1, 2, 3)]
            for v in range(4):
                dst[pl.ds(v * _L, _L)] = hvals[v]

            def _mid(blk, carry):
                t0 = 4 * _L + blk * (_BLK * _L)
                idxb = base + t0
                vals = [plsc.load_gather(src, [idxb + v * _L])
                        for v in range(_BLK)]
                for v in range(_BLK):
                    dst[pl.ds(t0 + v * _L, _L)] = vals[v]
                return carry

            lax.fori_loop(0, nblk, _mid, 0)

            idxl = base + (nvec - 1) * _L
            idxl = jnp.where(idxl > T - 1, 2 * (T - 1) - idxl, idxl)
            tvals = [plsc.load_gather(src, [base + (nvec - 4 + v) * _L])
                     for v in (0, 1, 2)] + [plsc.load_gather(src, [idxl])]
            for v in range(4):
                dst[pl.ds((nvec - 4 + v) * _L, _L)] = tvals[v]

            out_copy(r, j).start()

            @pl.when(loc + 2 < rows_per_w)
            def _():
                in_copy(r + 2, j).start()

        def group(g, carry):
            for j in range(2):
                loc = g * 2 + j
                do_row(row0 + loc, loc, j)
            return carry

        lax.fori_loop(0, rows_per_w // 2, group, 0)
        out_copy(row0 + rows_per_w - 2, 0).wait()
        out_copy(row0 + rows_per_w - 1, 1).wait()

    return body(x, shifts_i32)


def kernel(x, shifts):
    return _phase_shuffle_sc(x, shifts.astype(jnp.int32))
